# baseline (speedup 1.0000x reference)
import jax
import jax.numpy as jnp
from jax.experimental import pallas as pl
from jax.experimental.pallas import tpu as pltpu


def _fused_kernel(a_ref, x_ref, w1_ref, b1_ref, w2_ref, b2_ref,
                  wfc_ref, bfc_ref, o_ref, s1_ref, s2_ref):
    i = pl.program_id(0)
    nb = pl.num_programs(0) // 2
    rb = a_ref.shape[0]

    @pl.when(i == 0)
    def _sx():
        s1_ref[...] = jnp.dot(x_ref[...], w1_ref[...],
                              preferred_element_type=jnp.float32
                              ).astype(jnp.bfloat16)

    @pl.when(i < nb)
    def _layer1():
        h = jnp.dot(a_ref[...].astype(jnp.bfloat16), s1_ref[...],
                    preferred_element_type=jnp.float32)
        h = jnp.maximum(h + b1_ref[...], 0.0)
        s2_ref[pl.ds(i * rb, rb), :] = jnp.dot(
            h, w2_ref[...], preferred_element_type=jnp.float32)

    @pl.when(i >= nb)
    def _layer2():
        h = jnp.dot(a_ref[...].astype(jnp.bfloat16),
                    s2_ref[...].astype(jnp.bfloat16),
                    preferred_element_type=jnp.float32)
        h = h + b2_ref[...]
        logits = jax.lax.dot_general(
            h, wfc_ref[...], (((1,), (1,)), ((), ())),
            preferred_element_type=jnp.float32) + bfc_ref[...]
        m = jnp.max(logits, axis=1, keepdims=True)
        lse = jnp.log(jnp.sum(jnp.exp(logits - m), axis=1, keepdims=True))
        o_ref[...] = (logits - m) - lse


def kernel(x, adj, W1, b1, W2, b2, Wfc, bfc):
    n, nf = x.shape
    nh = W1.shape[1]
    nc = Wfc.shape[0]

    rb = 200
    nb = n // rb
    grid = (2 * nb,)

    out = pl.pallas_call(
        _fused_kernel,
        grid=grid,
        in_specs=[
            pl.BlockSpec((rb, n), lambda i: (i % nb, 0)),
            pl.BlockSpec((n, nf), lambda i: (0, 0)),
            pl.BlockSpec((nf, nh), lambda i: (0, 0)),
            pl.BlockSpec((1, nh), lambda i: (0, 0)),
            pl.BlockSpec((nh, nh), lambda i: (0, 0)),
            pl.BlockSpec((1, nh), lambda i: (0, 0)),
            pl.BlockSpec((nc, nh), lambda i: (0, 0)),
            pl.BlockSpec((1, nc), lambda i: (0, 0)),
        ],
        out_specs=pl.BlockSpec((rb, nc), lambda i: (i % nb, 0)),
        out_shape=jax.ShapeDtypeStruct((n, nc), jnp.float32),
        scratch_shapes=[pltpu.VMEM((n, nh), jnp.bfloat16),
                        pltpu.VMEM((n, nh), jnp.float32)],
        compiler_params=pltpu.CompilerParams(
            dimension_semantics=("arbitrary",)),
    )(adj, x, W1, b1.reshape(1, nh), W2, b2.reshape(1, nh),
      Wfc, bfc.reshape(1, nc))

    return out


# speedup vs baseline: 1.0619x; 1.0619x over previous
import jax
import jax.numpy as jnp
from jax.experimental import pallas as pl
from jax.experimental.pallas import tpu as pltpu


def _fused_kernel(a_ref, x_ref, w1_ref, b1_ref, w2_ref, b2_ref,
                  wfc_ref, bfc_ref, o_ref, s1_ref, s2_ref):
    i = pl.program_id(0)
    nb = pl.num_programs(0) // 2
    rb = a_ref.shape[0]

    @pl.when(i == 0)
    def _sx():
        s1_ref[...] = jnp.dot(x_ref[...], w1_ref[...],
                              preferred_element_type=jnp.float32
                              ).astype(jnp.bfloat16)

    @pl.when(i < nb)
    def _layer1():
        h = jnp.dot(a_ref[...].astype(jnp.bfloat16), s1_ref[...],
                    preferred_element_type=jnp.float32)
        h = jnp.maximum(h + b1_ref[...], 0.0)
        s2_ref[pl.ds(i * rb, rb), :] = jnp.dot(
            h, w2_ref[...], preferred_element_type=jnp.float32
            ).astype(jnp.bfloat16)

    @pl.when(i >= nb)
    def _layer2():
        h = jnp.dot(a_ref[...].astype(jnp.bfloat16), s2_ref[...],
                    preferred_element_type=jnp.float32)
        h = h + b2_ref[...]
        logits = jax.lax.dot_general(
            h, wfc_ref[...], (((1,), (1,)), ((), ())),
            preferred_element_type=jnp.float32) + bfc_ref[...]
        m = jnp.max(logits, axis=1, keepdims=True)
        lse = jnp.log(jnp.sum(jnp.exp(logits - m), axis=1, keepdims=True))
        o_ref[...] = (logits - m) - lse


def kernel(x, adj, W1, b1, W2, b2, Wfc, bfc):
    n, nf = x.shape
    nh = W1.shape[1]
    nc = Wfc.shape[0]

    rb = 400
    nb = n // rb
    grid = (2 * nb,)

    out = pl.pallas_call(
        _fused_kernel,
        grid=grid,
        in_specs=[
            pl.BlockSpec((rb, n), lambda i: (i % nb, 0)),
            pl.BlockSpec((n, nf), lambda i: (0, 0)),
            pl.BlockSpec((nf, nh), lambda i: (0, 0)),
            pl.BlockSpec((1, nh), lambda i: (0, 0)),
            pl.BlockSpec((nh, nh), lambda i: (0, 0)),
            pl.BlockSpec((1, nh), lambda i: (0, 0)),
            pl.BlockSpec((nc, nh), lambda i: (0, 0)),
            pl.BlockSpec((1, nc), lambda i: (0, 0)),
        ],
        out_specs=pl.BlockSpec((rb, nc), lambda i: (i % nb, 0)),
        out_shape=jax.ShapeDtypeStruct((n, nc), jnp.float32),
        scratch_shapes=[pltpu.VMEM((n, nh), jnp.bfloat16),
                        pltpu.VMEM((n, nh), jnp.bfloat16)],
        compiler_params=pltpu.CompilerParams(
            dimension_semantics=("arbitrary",)),
    )(adj, x, W1, b1.reshape(1, nh), W2, b2.reshape(1, nh),
      Wfc, bfc.reshape(1, nc))

    return out
